# baseline (speedup 1.0000x reference)
import jax
import jax.numpy as jnp
from jax import lax
from jax.experimental import pallas as pl
from jax.experimental.pallas import tpu as pltpu
from jax.experimental.pallas import tpu_sc as plsc

NUM_MONTH = 12
NUM_LOC = 100000
D = 128
B = 16384

_NC = 2
_NS = 16
_NW = _NC * _NS
_BPW = B // _NW
_CHUNK = 128
_NCHUNK = _BPW // _CHUNK


def _sc_gather_body(y_hbm, table_hbm, out_hbm, idx_v, rows_v, gsems, wsem):
    wid = lax.axis_index("s") * _NC + lax.axis_index("c")
    base = wid * _BPW
    pltpu.sync_copy(y_hbm.at[1, pl.ds(base, _BPW)], idx_v)
    copies = []
    for j in range(_NCHUNK):
        sl = pl.ds(j * _CHUNK, _CHUNK)
        copies.append(
            pltpu.async_copy(table_hbm.at[idx_v.at[sl]], rows_v.at[sl],
                             gsems.at[j])
        )
    writes = []
    for j in range(_NCHUNK):
        sl = pl.ds(j * _CHUNK, _CHUNK)
        copies[j].wait()
        writes.append(
            pltpu.async_copy(rows_v.at[sl],
                             out_hbm.at[pl.ds(base + j * _CHUNK, _CHUNK)],
                             wsem)
        )
    for w in writes:
        w.wait()


@jax.jit
def _sc_gather(loc_table, y):
    mesh = plsc.VectorSubcoreMesh(core_axis_name="c", subcore_axis_name="s")
    return pl.kernel(
        _sc_gather_body,
        out_type=jax.ShapeDtypeStruct((B, D), jnp.float32),
        mesh=mesh,
        scratch_types=[
            pltpu.VMEM((_BPW,), jnp.int32),
            pltpu.VMEM((_BPW, D), jnp.float32),
            pltpu.SemaphoreType.DMA((_NCHUNK,)),
            pltpu.SemaphoreType.DMA,
        ],
    )(y, loc_table)


_BB = 2048


def _mlp_body(month_ref, loc_ref, mt_ref, w1b_ref, b1_ref, w2_ref, b2_ref,
              out_ref):
    mt_proj = jnp.dot(mt_ref[...], w1b_ref[0], preferred_element_type=jnp.float32)
    labels = month_ref[0, 0, :]
    onehot = (labels[:, None]
              == lax.broadcasted_iota(jnp.int32, (_BB, NUM_MONTH), 1)
              ).astype(jnp.float32)
    h = (jnp.dot(onehot, mt_proj, preferred_element_type=jnp.float32)
         + jnp.dot(loc_ref[...], w1b_ref[1], preferred_element_type=jnp.float32)
         + b1_ref[...])
    h = h * jax.nn.sigmoid(h)
    out_ref[...] = (jnp.dot(h, w2_ref[...], preferred_element_type=jnp.float32)
                    + b2_ref[...])


@jax.jit
def _tc_mlp(y3, loc_embed, month_table, W1, b1, W2, b2):
    n_tiles = B // _BB
    w1_split = W1.reshape(2, D, D)
    return pl.pallas_call(
        _mlp_body,
        grid=(n_tiles,),
        in_specs=[
            pl.BlockSpec((1, 1, _BB), lambda i: (0, 0, i)),
            pl.BlockSpec((_BB, D), lambda i: (i, 0)),
            pl.BlockSpec(month_table.shape, lambda i: (0, 0)),
            pl.BlockSpec(w1_split.shape, lambda i: (0, 0, 0)),
            pl.BlockSpec((1, D), lambda i: (0, 0)),
            pl.BlockSpec((D, D), lambda i: (0, 0)),
            pl.BlockSpec((1, D), lambda i: (0, 0)),
        ],
        out_specs=pl.BlockSpec((_BB, D), lambda i: (i, 0)),
        out_shape=jax.ShapeDtypeStruct((B, D), jnp.float32),
    )(y3, loc_embed, month_table, w1_split, b1.reshape(1, D), W2,
      b2.reshape(1, D))


def kernel(y, month_table, loc_table, W1, b1, W2, b2):
    y = y.astype(jnp.int32)
    loc_embed = _sc_gather(loc_table, y)
    y3 = y.reshape(2, 1, B)
    return _tc_mlp(y3, loc_embed, month_table, W1, b1, W2, b2)
